# Initial kernel scaffold; baseline (speedup 1.0000x reference)
import jax
import jax.numpy as jnp
from jax.experimental import pallas as pl


def kernel(x, spatial_edge_index, temporal_edge_index, sWl1, sWr1, sb1, sWl2, sWr2, sb2, tWl1, tWr1, tb1, tWl2, tWr2, tb2, spW1, spb1, spW2, spb2, tpW1, tpb1, tpW2, tpb2):
    raise NotImplementedError("write your pallas kernel here")


# speedup vs baseline: 1.6491x; 1.6491x over previous
import functools

import jax
import jax.numpy as jnp
from jax import lax
from jax.experimental import pallas as pl
from jax.experimental.pallas import tpu as pltpu
from jax.experimental.pallas import tpu_sc as plsc

N = 10000
C = 128
L = 20
NW = 3
NT = N * NW
E = 320000
H = 64
LANES = 128
NTILES = 16
NSC = 2
WCNT = 16


def _rup(a, m):
    return (a + m - 1) // m * m


NP_S = _rup(N + 1, 128)
NP_T = _rup(NT + 1, 128)
CH_SEG = _rup(E // NTILES, LANES) // LANES
CH_CNT = _rup(E // (NTILES * NSC), LANES) // LANES


@functools.lru_cache(maxsize=None)
def _seg_sum_kernel(n_pad, nchunks):
    mesh = plsc.VectorSubcoreMesh(core_axis_name="c", subcore_axis_name="s")
    rpt = n_pad // NTILES

    def body(src_hbm, dst_hbm, table_hbm, zeros_hbm, out_hbm,
             src_v, dst_v, rows_v, acc_sh, sem):
        c = lax.axis_index("c")
        s = lax.axis_index("s")
        pltpu.sync_copy(zeros_hbm.at[pl.ds(s * rpt, rpt)],
                        acc_sh.at[pl.ds(s * rpt, rpt)])
        plsc.subcore_barrier()

        def step(i, carry):
            pltpu.sync_copy(src_hbm.at[c, s, i], src_v)
            pltpu.sync_copy(dst_hbm.at[s, i], dst_v)
            pltpu.async_copy(table_hbm.at[src_v], rows_v, sem).wait()
            pltpu.sync_copy(rows_v, acc_sh.at[dst_v], add=True)
            return carry

        lax.fori_loop(0, nchunks, step, 0)
        plsc.subcore_barrier()
        pltpu.sync_copy(acc_sh.at[pl.ds(s * rpt, rpt)],
                        out_hbm.at[c, pl.ds(s * rpt, rpt)])

    return pl.kernel(
        body,
        out_type=jax.ShapeDtypeStruct((NSC, n_pad, H), jnp.float32),
        mesh=mesh,
        compiler_params=pltpu.CompilerParams(use_tc_tiling_on_sc=False),
        scratch_types=[
            pltpu.VMEM((LANES,), jnp.int32),
            pltpu.VMEM((LANES,), jnp.int32),
            pltpu.VMEM((LANES, H), jnp.float32),
            pltpu.VMEM_SHARED((n_pad, H), jnp.float32),
            pltpu.SemaphoreType.DMA,
        ],
    )


@functools.lru_cache(maxsize=None)
def _seg_cnt_kernel(n_pad):
    mesh = plsc.VectorSubcoreMesh(core_axis_name="c", subcore_axis_name="s")
    rpt = n_pad // NTILES

    def body(dst_hbm, ones_hbm, zeros_hbm, out_hbm, dst_v, ones_v, acc_sh):
        c = lax.axis_index("c")
        s = lax.axis_index("s")
        pltpu.sync_copy(zeros_hbm.at[pl.ds(s * rpt, rpt)],
                        acc_sh.at[pl.ds(s * rpt, rpt)])
        pltpu.sync_copy(ones_hbm, ones_v)
        plsc.subcore_barrier()

        def step(i, carry):
            pltpu.sync_copy(dst_hbm.at[c, s, i], dst_v)
            pltpu.sync_copy(ones_v, acc_sh.at[dst_v], add=True)
            return carry

        lax.fori_loop(0, CH_CNT, step, 0)
        plsc.subcore_barrier()
        pltpu.sync_copy(acc_sh.at[pl.ds(s * rpt, rpt)],
                        out_hbm.at[c, pl.ds(s * rpt, rpt)])

    return pl.kernel(
        body,
        out_type=jax.ShapeDtypeStruct((NSC, n_pad, WCNT), jnp.float32),
        mesh=mesh,
        compiler_params=pltpu.CompilerParams(use_tc_tiling_on_sc=False),
        scratch_types=[
            pltpu.VMEM((LANES,), jnp.int32),
            pltpu.VMEM((LANES, WCNT), jnp.float32),
            pltpu.VMEM_SHARED((n_pad, WCNT), jnp.float32),
        ],
    )


def _prep_seg_edges(src, dst, rows, n_dummy):
    per = NTILES * CH_SEG * LANES
    pad = per - E
    src_p = jnp.concatenate([src, jnp.zeros((pad,), jnp.int32)])
    dst_p = jnp.concatenate([dst, jnp.full((pad,), n_dummy, jnp.int32)])
    src2 = jnp.stack([src_p, src_p + rows]).reshape(NSC, NTILES, CH_SEG, LANES)
    dst2 = dst_p.reshape(NTILES, CH_SEG, LANES)
    return src2, dst2


def _prep_cnt_edges(dst, n_dummy):
    per = NSC * NTILES * CH_CNT * LANES
    pad = per - E
    dst_p = jnp.concatenate([dst, jnp.full((pad,), n_dummy, jnp.int32)])
    return dst_p.reshape(NSC, NTILES, CH_CNT, LANES)


def _means(x):
    B = 200

    def body(x_ref, sp_ref, fl_ref):
        xb = x_ref[...]
        w0 = jnp.sum(xb[:, :, 0:10], axis=2) * 0.1
        w1 = jnp.sum(xb[:, :, 5:15], axis=2) * 0.1
        w2 = jnp.sum(xb[:, :, 10:20], axis=2) * 0.1
        spin = (w0 + w2) * 0.5
        sp_ref[0] = spin[:, :H]
        sp_ref[1] = spin[:, H:]
        fl_ref[0, 0] = w0[:, :H]
        fl_ref[0, 1] = w1[:, :H]
        fl_ref[0, 2] = w2[:, :H]
        fl_ref[1, 0] = w0[:, H:]
        fl_ref[1, 1] = w1[:, H:]
        fl_ref[1, 2] = w2[:, H:]

    return pl.pallas_call(
        body,
        grid=(N // B,),
        in_specs=[pl.BlockSpec((B, C, L), lambda i: (i, 0, 0))],
        out_specs=[pl.BlockSpec((2, B, H), lambda i: (0, i, 0)),
                   pl.BlockSpec((2, NW, B, H), lambda i: (0, 0, i, 0))],
        out_shape=[jax.ShapeDtypeStruct((2, N, H), jnp.float32),
                   jax.ShapeDtypeStruct((2, NW, N, H), jnp.float32)],
    )(x)


def _layer(agg, cnt, xin, Wl, Wr, b, *, relu, normalize, halves):
    n = agg.shape[1]
    B = 1000
    dout = Wl.shape[1]

    def body(agg_ref, cnt_ref, xin_ref, wl_ref, wr_ref, b_ref, out_ref):
        a = jnp.concatenate([agg_ref[0], agg_ref[1]], axis=1)
        ct = cnt_ref[0][:, 0:1] + cnt_ref[1][:, 0:1]
        a = a / jnp.maximum(ct, 1.0)
        xi = jnp.concatenate([xin_ref[0], xin_ref[1]], axis=1)
        y = (jnp.dot(a, wl_ref[...], preferred_element_type=jnp.float32)
             + jnp.dot(xi, wr_ref[...], preferred_element_type=jnp.float32)
             + b_ref[...])
        if relu:
            y = jnp.maximum(y, 0.0)
        if normalize:
            nm = jnp.sqrt(jnp.sum(y * y, axis=1, keepdims=True))
            y = y / jnp.maximum(nm, 1e-12)
        if halves:
            out_ref[0] = y[:, :H]
            out_ref[1] = y[:, H:]
        else:
            out_ref[...] = y

    if halves:
        out_spec = pl.BlockSpec((2, B, H), lambda i: (0, i, 0))
        out_shape = jax.ShapeDtypeStruct((2, n, H), jnp.float32)
    else:
        out_spec = pl.BlockSpec((B, dout), lambda i: (i, 0))
        out_shape = jax.ShapeDtypeStruct((n, dout), jnp.float32)

    return pl.pallas_call(
        body,
        grid=(n // B,),
        in_specs=[
            pl.BlockSpec((2, B, H), lambda i: (0, i, 0)),
            pl.BlockSpec((2, B, WCNT), lambda i: (0, i, 0)),
            pl.BlockSpec((2, B, H), lambda i: (0, i, 0)),
            pl.BlockSpec(Wl.shape, lambda i: (0, 0)),
            pl.BlockSpec(Wr.shape, lambda i: (0, 0)),
            pl.BlockSpec((1, dout), lambda i: (0, 0)),
        ],
        out_specs=out_spec,
        out_shape=out_shape,
    )(agg, cnt, xin, Wl, Wr, b.reshape(1, dout))


def _proj(tin, W1, b1, W2, b2):
    nw = tin.shape[0]
    B = 1000
    fd = W1.shape[1]

    def body(t_ref, w1_ref, b1_ref, w2_ref, b2_ref, out_ref):
        t = jnp.mean(t_ref[...], axis=0)
        h1 = jnp.maximum(
            jnp.dot(t, w1_ref[...], preferred_element_type=jnp.float32)
            + b1_ref[...], 0.0)
        y = (jnp.dot(h1, w2_ref[...], preferred_element_type=jnp.float32)
             + b2_ref[...])
        nm = jnp.sqrt(jnp.sum(y * y, axis=1, keepdims=True))
        out_ref[...] = y / jnp.maximum(nm, 1e-12)

    return pl.pallas_call(
        body,
        grid=(N // B,),
        in_specs=[
            pl.BlockSpec((nw, B, tin.shape[2]), lambda i: (0, i, 0)),
            pl.BlockSpec(W1.shape, lambda i: (0, 0)),
            pl.BlockSpec((1, fd), lambda i: (0, 0)),
            pl.BlockSpec(W2.shape, lambda i: (0, 0)),
            pl.BlockSpec((1, W2.shape[1]), lambda i: (0, 0)),
        ],
        out_specs=pl.BlockSpec((B, W2.shape[1]), lambda i: (i, 0)),
        out_shape=jax.ShapeDtypeStruct((N, W2.shape[1]), jnp.float32),
    )(tin, W1, b1.reshape(1, fd), W2, b2.reshape(1, W2.shape[1]))


def kernel(x, spatial_edge_index, temporal_edge_index,
           sWl1, sWr1, sb1, sWl2, sWr2, sb2,
           tWl1, tWr1, tb1, tWl2, tWr2, tb2,
           spW1, spb1, spW2, spb2, tpW1, tpb1, tpW2, tpb2):
    zeros_s = jnp.zeros((NP_S, H), jnp.float32)
    zeros_t = jnp.zeros((NP_T, H), jnp.float32)
    zeros_cs = jnp.zeros((NP_S, WCNT), jnp.float32)
    zeros_ct = jnp.zeros((NP_T, WCNT), jnp.float32)
    ones_v = jnp.ones((LANES, WCNT), jnp.float32)

    sp_h, fl4 = _means(x)
    fl_h = fl4.reshape(2, NT, H)

    s_src2, s_dst2 = _prep_seg_edges(spatial_edge_index[0],
                                     spatial_edge_index[1], N, N)
    tei = (temporal_edge_index % NW) * N + temporal_edge_index // NW
    t_src2, t_dst2 = _prep_seg_edges(tei[0], tei[1], NT, NT)
    s_cnt_dst = _prep_cnt_edges(spatial_edge_index[1], N)
    t_cnt_dst = _prep_cnt_edges(tei[1], NT)

    cnt_s = _seg_cnt_kernel(NP_S)(s_cnt_dst, ones_v, zeros_cs)[:, :N]
    cnt_t = _seg_cnt_kernel(NP_T)(t_cnt_dst, ones_v, zeros_ct)[:, :NT]

    seg_s = _seg_sum_kernel(NP_S, CH_SEG)
    seg_t = _seg_sum_kernel(NP_T, CH_SEG)

    agg1 = seg_s(s_src2, s_dst2, sp_h.reshape(2 * N, H), zeros_s)[:, :N]
    hs = _layer(agg1, cnt_s, sp_h, sWl1, sWr1, sb1,
                relu=True, normalize=False, halves=True)
    agg2 = seg_s(s_src2, s_dst2, hs.reshape(2 * N, H), zeros_s)[:, :N]
    sp_emb = _layer(agg2, cnt_s, hs, sWl2, sWr2, sb2,
                    relu=False, normalize=True, halves=False)
    sp_proj = _proj(sp_emb.reshape(1, N, sp_emb.shape[1]),
                    spW1, spb1, spW2, spb2)

    aggt1 = seg_t(t_src2, t_dst2, fl_h.reshape(2 * NT, H), zeros_t)[:, :NT]
    ht = _layer(aggt1, cnt_t, fl_h, tWl1, tWr1, tb1,
                relu=True, normalize=False, halves=True)
    aggt2 = seg_t(t_src2, t_dst2, ht.reshape(2 * NT, H), zeros_t)[:, :NT]
    t_emb = _layer(aggt2, cnt_t, ht, tWl2, tWr2, tb2,
                   relu=False, normalize=True, halves=False)
    t_proj = _proj(t_emb.reshape(NW, N, t_emb.shape[1]),
                   tpW1, tpb1, tpW2, tpb2)

    return sp_proj, t_proj
